# baseline (speedup 1.0000x reference)
import functools

import jax
import jax.numpy as jnp
from jax import lax
from jax.experimental import pallas as pl
from jax.experimental.pallas import tpu as pltpu
from jax.experimental.pallas import tpu_sc as plsc

B = 16384
D = 64
NC = 2
NS = 16
NW = NC * NS
BPW = B // NW
CH = 128
NCH = BPW // CH


def _sc_body(uidx, iidx, gidx, midx, ue, ie, ge, me, ub, ib, gb, mb,
             partials, bias_out,
             uidx_v, iidx_v, gidx_v, midx_v, arows, brows,
             ubv, ibv, gbv, mbv, bsum_v, accv, sem_r, sem_b):
    wid = lax.axis_index("s") * NC + lax.axis_index("c")

    pltpu.sync_copy(uidx.at[wid], uidx_v)
    pltpu.sync_copy(iidx.at[wid], iidx_v)
    pltpu.sync_copy(gidx.at[wid], gidx_v)
    pltpu.sync_copy(midx.at[wid], midx_v)

    bias_descs = []
    for tbl, dst, idx in ((ub, ubv, uidx_v), (ib, ibv, iidx_v),
                          (gb, gbv, gidx_v), (mb, mbv, midx_v)):
        for c in range(NCH):
            bias_descs.append(
                pltpu.async_copy(tbl.at[idx.at[c]],
                                 dst.at[pl.ds(c * CH, CH)], sem_b))

    def fire_rows(tbl_a, idx_a, tbl_b, idx_b):
        descs = []
        for c in range(NCH):
            descs.append(pltpu.async_copy(
                tbl_a.at[idx_a.at[c]], arows.at[pl.ds(c * CH, CH), :], sem_r))
            descs.append(pltpu.async_copy(
                tbl_b.at[idx_b.at[c]], brows.at[pl.ds(c * CH, CH), :], sem_r))
        return descs

    def accum(acc):
        def body(r, acc):
            for k in range(D // 16):
                s = pl.ds(k * 16, 16)
                acc = acc + arows[r, s] * brows[r, s]
            return acc
        return lax.fori_loop(0, BPW, body, acc)

    descs = fire_rows(ue, uidx_v, ie, iidx_v)
    for d in descs:
        d.wait()
    acc = accum(jnp.zeros((16,), jnp.float32))

    descs = fire_rows(me, midx_v, ge, gidx_v)
    for d in descs:
        d.wait()
    acc = accum(acc)

    accv[...] = acc
    pltpu.sync_copy(accv, partials.at[wid])

    for d in bias_descs:
        d.wait()

    def bbody(j, carry):
        s = pl.ds(j * 16, 16)
        bsum_v[s] = ubv[s] + ibv[s] + gbv[s] + mbv[s]
        return carry
    lax.fori_loop(0, BPW // 16, bbody, 0)
    pltpu.sync_copy(bsum_v, bias_out.at[pl.ds(wid * BPW, BPW)])


_sc_call = functools.partial(
    pl.kernel,
    out_type=(jax.ShapeDtypeStruct((NW, 16), jnp.float32),
              jax.ShapeDtypeStruct((B,), jnp.float32)),
    mesh=plsc.VectorSubcoreMesh(core_axis_name="c", subcore_axis_name="s"),
    compiler_params=pltpu.CompilerParams(use_tc_tiling_on_sc=False),
    scratch_types=[
        pltpu.VMEM((NCH, CH), jnp.int32),
        pltpu.VMEM((NCH, CH), jnp.int32),
        pltpu.VMEM((NCH, CH), jnp.int32),
        pltpu.VMEM((NCH, CH), jnp.int32),
        pltpu.VMEM((BPW, D), jnp.float32),
        pltpu.VMEM((BPW, D), jnp.float32),
        pltpu.VMEM((BPW,), jnp.float32),
        pltpu.VMEM((BPW,), jnp.float32),
        pltpu.VMEM((BPW,), jnp.float32),
        pltpu.VMEM((BPW,), jnp.float32),
        pltpu.VMEM((BPW,), jnp.float32),
        pltpu.VMEM((16,), jnp.float32),
        pltpu.SemaphoreType.DMA,
        pltpu.SemaphoreType.DMA,
    ],
)(_sc_body)


def _combine_body(p_ref, b_ref, o_ref):
    o_ref[...] = b_ref[...] + jnp.sum(p_ref[...])


def kernel(inputs, user_emb, user_bias, item_emb, item_bias,
           genre_emb, genre_bias, month_emb, month_bias):
    uidx = inputs[:, 0].reshape(NW, NCH, CH)
    iidx = inputs[:, 1].reshape(NW, NCH, CH)
    gidx = inputs[:, 2].reshape(NW, NCH, CH)
    midx = inputs[:, 3].reshape(NW, NCH, CH)

    partials, bias_sum = _sc_call(
        uidx, iidx, gidx, midx,
        user_emb, item_emb, genre_emb, month_emb,
        user_bias.reshape(-1), item_bias.reshape(-1),
        genre_bias.reshape(-1), month_bias.reshape(-1))

    out = pl.pallas_call(
        _combine_body,
        out_shape=jax.ShapeDtypeStruct((128, 128), jnp.float32),
    )(partials, bias_sum.reshape(128, 128))
    return out.reshape(B, 1)


# speedup vs baseline: 1.0909x; 1.0909x over previous
import functools

import jax
import jax.numpy as jnp
from jax import lax
from jax.experimental import pallas as pl
from jax.experimental.pallas import tpu as pltpu
from jax.experimental.pallas import tpu_sc as plsc

B = 16384
D = 64
NC = 2
NS = 16
NW = NC * NS
BPW = B // NW
CH = 128
NCH = BPW // CH


def _sc_body(uidx, iidx, gidx, midx, ue, ie, ge, me, ub, ib, gb, mb,
             partials, bias_out,
             uv2, iv2, gv2, mv2, ud2, id2,
             urows, irows, ubb, ibb, gbb, mbb,
             ge_t, me_t, bsum_v, accv,
             sem_a, sem_b, sem_c):
    wid = lax.axis_index("s") * NC + lax.axis_index("c")

    pltpu.sync_copy(uidx.at[wid], uv2)
    pltpu.sync_copy(iidx.at[wid], iv2)
    pltpu.sync_copy(gidx.at[wid], gv2)
    pltpu.sync_copy(midx.at[wid], mv2)
    pltpu.sync_copy(ge, ge_t)
    pltpu.sync_copy(me, me_t)

    for j in range(NCH):
        for l in range(8):
            s = pl.ds(l * 16, 16)
            ud2[j, s] = lax.shift_right_logical(uv2[j, s], 1)
            id2[j, s] = lax.shift_right_logical(iv2[j, s], 1)

    bias_descs = []
    for j in range(NCH):
        d = pl.ds(j * CH, CH)
        bias_descs += [
            pltpu.async_copy(ub.at[uv2.at[j]], ubb.at[d], sem_c),
            pltpu.async_copy(ib.at[iv2.at[j]], ibb.at[d], sem_c),
            pltpu.async_copy(gb.at[gv2.at[j]], gbb.at[d], sem_c),
            pltpu.async_copy(mb.at[mv2.at[j]], mbb.at[d], sem_c),
        ]

    def fire(c, buf):
        sem = sem_a if buf == 0 else sem_b
        return [
            pltpu.async_copy(ue.at[ud2.at[c]], urows.at[buf], sem),
            pltpu.async_copy(ie.at[id2.at[c]], irows.at[buf], sem),
        ]

    acc = jnp.zeros((16,), jnp.float32)
    descs = fire(0, 0)
    for c in range(NCH):
        buf = c % 2
        nxt = fire(c + 1, 1 - buf) if c + 1 < NCH else []
        for d_ in descs:
            d_.wait()
        descs = nxt

        def grp_body(jj, a, c=c, buf=buf):
            base = jj * 16
            s = pl.ds(base, 16)
            u16 = uv2[c, s]
            i16 = iv2[c, s]
            g16 = gv2[c, s]
            m16 = mv2[c, s]
            uoffs = (u16 & 1) * 64
            ioffs = (i16 & 1) * 64
            goffs = (g16 & 1) * 64
            moffs = (m16 & 1) * 64
            grows = lax.shift_right_logical(g16, 1)
            mrows = lax.shift_right_logical(m16, 1)
            for l in range(16):
                r = base + l
                uo = uoffs[l]
                io = ioffs[l]
                gr = grows[l]
                go = goffs[l]
                mr = mrows[l]
                mo = moffs[l]
                for k in range(4):
                    uu = urows[buf, r, pl.ds(uo + 16 * k, 16)]
                    ii = irows[buf, r, pl.ds(io + 16 * k, 16)]
                    gg = ge_t[gr, pl.ds(go + 16 * k, 16)]
                    mm = me_t[mr, pl.ds(mo + 16 * k, 16)]
                    a = a + uu * ii + gg * mm
            return a

        acc = lax.fori_loop(0, CH // 16, grp_body, acc)

    for d_ in bias_descs:
        d_.wait()
    for j in range(BPW // 16):
        s = pl.ds(j * 16, 16)
        bsum_v[s] = ubb[s] + ibb[s] + gbb[s] + mbb[s]

    accv[...] = acc
    pltpu.sync_copy(accv, partials.at[wid])
    pltpu.sync_copy(bsum_v, bias_out.at[pl.ds(wid * BPW, BPW)])


_sc_call = functools.partial(
    pl.kernel,
    out_type=(jax.ShapeDtypeStruct((NW, 16), jnp.float32),
              jax.ShapeDtypeStruct((B,), jnp.float32)),
    mesh=plsc.VectorSubcoreMesh(core_axis_name="c", subcore_axis_name="s"),
    compiler_params=pltpu.CompilerParams(use_tc_tiling_on_sc=True),
    scratch_types=[
        pltpu.VMEM((NCH, CH), jnp.int32),
        pltpu.VMEM((NCH, CH), jnp.int32),
        pltpu.VMEM((NCH, CH), jnp.int32),
        pltpu.VMEM((NCH, CH), jnp.int32),
        pltpu.VMEM((NCH, CH), jnp.int32),
        pltpu.VMEM((NCH, CH), jnp.int32),
        pltpu.VMEM((2, CH, 128), jnp.float32),
        pltpu.VMEM((2, CH, 128), jnp.float32),
        pltpu.VMEM((BPW,), jnp.float32),
        pltpu.VMEM((BPW,), jnp.float32),
        pltpu.VMEM((BPW,), jnp.float32),
        pltpu.VMEM((BPW,), jnp.float32),
        pltpu.VMEM((10, 128), jnp.float32),
        pltpu.VMEM((6, 128), jnp.float32),
        pltpu.VMEM((BPW,), jnp.float32),
        pltpu.VMEM((16,), jnp.float32),
        pltpu.SemaphoreType.DMA,
        pltpu.SemaphoreType.DMA,
        pltpu.SemaphoreType.DMA,
    ],
)(_sc_body)


def _combine_body(p_ref, b_ref, o_ref):
    o_ref[...] = b_ref[...] + jnp.sum(p_ref[...])


def kernel(inputs, user_emb, user_bias, item_emb, item_bias,
           genre_emb, genre_bias, month_emb, month_bias):
    uidx = inputs[:, 0].reshape(NW, NCH, CH)
    iidx = inputs[:, 1].reshape(NW, NCH, CH)
    gidx = inputs[:, 2].reshape(NW, NCH, CH)
    midx = inputs[:, 3].reshape(NW, NCH, CH)

    partials, bias_sum = _sc_call(
        uidx, iidx, gidx, midx,
        user_emb.reshape(-1, 128), item_emb.reshape(-1, 128),
        genre_emb.reshape(-1, 128), month_emb.reshape(-1, 128),
        user_bias.reshape(-1), item_bias.reshape(-1),
        genre_bias.reshape(-1), month_bias.reshape(-1))

    out = pl.pallas_call(
        _combine_body,
        out_shape=jax.ShapeDtypeStruct((128, 128), jnp.float32),
    )(partials, bias_sum.reshape(128, 128))
    return out.reshape(B, 1)
